# baseline (speedup 1.0000x reference)
import functools

import jax
import jax.numpy as jnp
from jax import lax
from jax.experimental import pallas as pl
from jax.experimental.pallas import tpu as pltpu
from jax.experimental.pallas import tpu_sc as plsc

_L = 16
_NC, _NS = 2, 16
_NW = _NC * _NS
_RPW = 32
_KC = 1280
_NCHUNK = 78
_TAIL = 160
_UNROLL = 2


def _mv_body(x_hbm, w_hbm, o_hbm, xbuf, wbuf, xtail, wtail, obuf,
             xsem, wsem, tsem, twsem):
    wid = lax.axis_index("c") * _NS + lax.axis_index("s")
    row0 = wid * _RPW

    def cp_x(c, b):
        return pltpu.make_async_copy(
            x_hbm.at[pl.ds(row0, _RPW), pl.ds(c * _KC, _KC)],
            xbuf.at[b], xsem.at[b])

    def cp_w(c, b):
        return pltpu.make_async_copy(
            w_hbm.at[pl.ds(c * _KC, _KC)], wbuf.at[b], wsem.at[b])

    cp_xt = pltpu.make_async_copy(
        x_hbm.at[pl.ds(row0, _RPW), pl.ds(_NCHUNK * _KC, _TAIL)],
        xtail, tsem)
    cp_wt = pltpu.make_async_copy(
        w_hbm.at[pl.ds(_NCHUNK * _KC, _TAIL)], wtail, twsem)

    cp_x(0, 0).start()
    cp_w(0, 0).start()
    cp_x(1, 1).start()
    cp_w(1, 1).start()
    cp_xt.start()
    cp_wt.start()

    def chunk_accum(xs, ws, nvec, accs):
        def kv_body(kv, accs):
            out = list(accs)
            for u in range(_UNROLL):
                off = (kv * _UNROLL + u) * _L
                wv = ws[pl.ds(off, _L)]
                for r in range(_RPW):
                    out[r] = out[r] + xs[r, pl.ds(off, _L)] * wv
            return tuple(out)
        return lax.fori_loop(0, nvec // _UNROLL, kv_body, accs)

    def half(c, b, accs):
        cp_x(c, b).wait()
        cp_w(c, b).wait()
        accs = chunk_accum(xbuf.at[b], wbuf.at[b], _KC // _L, accs)

        @pl.when(c + 2 < _NCHUNK)
        def _():
            cp_x(c + 2, b).start()
            cp_w(c + 2, b).start()

        return accs

    def outer(cc, accs):
        accs = half(2 * cc, 0, accs)
        accs = half(2 * cc + 1, 1, accs)
        return accs

    accs = tuple(jnp.zeros((_L,), jnp.float32) for _ in range(_RPW))
    accs = lax.fori_loop(0, _NCHUNK // 2, outer, accs)

    cp_xt.wait()
    cp_wt.wait()
    accs = chunk_accum(xtail, wtail, _TAIL // _L, accs)

    lane = lax.iota(jnp.int32, _L)
    dnums = lax.GatherDimensionNumbers(
        offset_dims=(), collapsed_slice_dims=(0,), start_index_map=(0,))

    def lane_sum(v):
        for sh in (8, 4, 2, 1):
            perm = lax.gather(
                v, (lane ^ sh)[:, None], dnums, (1,),
                mode=lax.GatherScatterMode.PROMISE_IN_BOUNDS)
            v = v + perm
        return v

    for g in range(_RPW // _L):
        res = jnp.zeros((_L,), jnp.float32)
        for r in range(_L):
            res = jnp.where(lane == r, lane_sum(accs[g * _L + r]), res)
        obuf[pl.ds(g * _L, _L)] = res
    pltpu.sync_copy(obuf, o_hbm.at[pl.ds(row0, _RPW)])


@jax.jit
def kernel(x, weight, bias):
    batch, _ = x.shape
    mesh = plsc.VectorSubcoreMesh(core_axis_name="c", subcore_axis_name="s")
    run = pl.kernel(
        _mv_body,
        mesh=mesh,
        out_type=jax.ShapeDtypeStruct((batch,), jnp.float32),
        scratch_types=[
            pltpu.VMEM((2, _RPW, _KC), jnp.float32),
            pltpu.VMEM((2, _KC), jnp.float32),
            pltpu.VMEM((_RPW, _TAIL), jnp.float32),
            pltpu.VMEM((_TAIL,), jnp.float32),
            pltpu.VMEM((_RPW,), jnp.float32),
            pltpu.SemaphoreType.DMA((2,)),
            pltpu.SemaphoreType.DMA((2,)),
            pltpu.SemaphoreType.DMA,
            pltpu.SemaphoreType.DMA,
        ],
    )
    out = run(x, weight.reshape(-1))
    return out.reshape(batch, 1) + bias


# speedup vs baseline: 1.1872x; 1.1872x over previous
import functools

import jax
import jax.numpy as jnp
from jax.experimental import pallas as pl
from jax.experimental.pallas import tpu as pltpu

_BB = 256
_NB = 4
_KB = 2048


def _mv_kernel(x0, x1, x2, x3, w_ref, b_ref, o_ref, a0, a1, a2, a3,
               *, vocab, nk):
    k = pl.program_id(0)
    xs = (x0, x1, x2, x3)
    accs = (a0, a1, a2, a3)

    def band(j):
        blk = xs[j][...]
        return blk.reshape(_BB, _KB)

    @pl.when(k == 0)
    def _init():
        for a in accs:
            a[...] = jnp.zeros_like(a)

    @pl.when(k < nk - 1)
    def _body():
        wc = w_ref[...]
        for j, a in enumerate(accs):
            a[...] += band(j) * wc

    @pl.when(k == nk - 1)
    def _tail():
        col = jax.lax.broadcasted_iota(jnp.int32, (1, _KB), 1)
        valid = col + k * _KB < vocab
        wc = w_ref[...]
        for j, a in enumerate(accs):
            a[...] += jnp.where(valid, band(j), 0.0) * wc
            o_ref[pl.ds(j * _BB, _BB), :] = (
                jnp.sum(a[...], axis=1, keepdims=True) + b_ref[0, 0]
            )


@jax.jit
def kernel(x, weight, bias):
    batch, vocab = x.shape
    nk = pl.cdiv(vocab, _KB)
    wpad = jnp.pad(weight.reshape(-1), (0, nk * _KB - vocab))

    v0 = x
    v1 = x.reshape(1, batch, vocab)
    v2 = x.reshape(2, batch // 2, vocab)
    v3 = x.reshape(4, batch // 4, vocab)

    out = pl.pallas_call(
        functools.partial(_mv_kernel, vocab=vocab, nk=nk),
        grid=(nk,),
        in_specs=[
            pl.BlockSpec((_BB, _KB), lambda k: (0, k)),
            pl.BlockSpec((1, _BB, _KB), lambda k: (0, 1, k)),
            pl.BlockSpec((1, _BB, _KB), lambda k: (1, 0, k)),
            pl.BlockSpec((1, _BB, _KB), lambda k: (3, 0, k)),
            pl.BlockSpec((1, _KB), lambda k: (0, k)),
            pl.BlockSpec((1, 1), lambda k: (0, 0)),
        ],
        out_specs=pl.BlockSpec((batch, 1), lambda k: (0, 0)),
        out_shape=jax.ShapeDtypeStruct((batch, 1), jnp.float32),
        scratch_shapes=[pltpu.VMEM((_BB, _KB), jnp.float32)
                        for _ in range(_NB)],
        compiler_params=pltpu.CompilerParams(
            dimension_semantics=("arbitrary",)
        ),
    )(v0, v1, v2, v3, wpad.reshape(1, -1), bias.reshape(1, 1))
    return out
